# baseline (speedup 1.0000x reference)
import functools

import jax
import jax.numpy as jnp
from jax import lax
from jax.experimental import pallas as pl
from jax.experimental.pallas import tpu as pltpu
from jax.experimental.pallas import tpu_sc as plsc

_GS = 128
_LANES = 16
_VPG = _GS // _LANES
_KG = 16
_U = 6


def _sort16(v):
    return lax.sort(v, dimension=0)


def _rev(v):
    return lax.rev(v, (0,))


def _merge(a, b):
    m = len(a)
    c = a + [_rev(b[m - 1 - i]) for i in range(m)]
    stride = m
    while stride >= 1:
        nxt = list(c)
        for base in range(0, 2 * m, 2 * stride):
            for i in range(stride):
                lo, hi = c[base + i], c[base + stride + i]
                nxt[base + i] = jnp.minimum(lo, hi)
                nxt[base + stride + i] = jnp.maximum(lo, hi)
        c = nxt
        stride //= 2
    return [_sort16(v) for v in c]


def _sort_group(vs):
    runs = [[_sort16(v)] for v in vs]
    while len(runs) > 1:
        runs = [_merge(runs[2 * i], runs[2 * i + 1])
                for i in range(len(runs) // 2)]
    return runs[0]


@functools.lru_cache(maxsize=None)
def _build(nbatch, nfeat):
    ngroups = nfeat // _GS
    info = plsc.get_sparse_core_info()
    nc = info.num_cores
    gpw = ngroups // 2
    nch = gpw // _KG
    mesh = plsc.VectorSubcoreMesh(core_axis_name="c", subcore_axis_name="s")

    w = _GS * _KG

    @functools.partial(
        pl.kernel,
        mesh=mesh,
        out_type=jax.ShapeDtypeStruct((nbatch, nfeat), jnp.float32),
        scratch_types=[
            pltpu.VMEM((8, w), jnp.float32),
            pltpu.VMEM((8, w), jnp.float32),
            pltpu.VMEM((8, w), jnp.float32),
            pltpu.VMEM((8, w), jnp.float32),
            pltpu.SemaphoreType.DMA,
            pltpu.SemaphoreType.DMA,
            pltpu.SemaphoreType.DMA,
            pltpu.SemaphoreType.DMA,
        ],
        compiler_params=pltpu.CompilerParams(needs_layout_passes=False),
    )
    def sc_group_sort(x_hbm, out_hbm, ib0, ib1, ob0, ob1, si0, si1, so0, so1):
        wid = lax.axis_index("s") * nc + lax.axis_index("c")
        trow = wid // 2
        ghalf = wid % 2
        r0 = trow * 8
        g0 = ghalf * gpw

        def src(c):
            return x_hbm.at[pl.ds(r0, 8), pl.ds((g0 + c * _KG) * _GS, w)]

        def dst(c):
            return out_hbm.at[pl.ds(r0, 8), pl.ds((g0 + c * _KG) * _GS, w)]

        def compute(ib, ob):
            @plsc.parallel_loop(0, 8 * _KG, step=1, unroll=_U)
            def body(idx):
                br = lax.rem(idx, 8)
                col = (idx // 8) * _GS
                vs = [ib[br, pl.ds(col + j * _LANES, _LANES)]
                      for j in range(_VPG)]
                sv = _sort_group(vs)
                for j in range(_VPG):
                    ob[br, pl.ds(col + j * _LANES, _LANES)] = sv[j]

        pltpu.async_copy(src(0), ib0, si0)
        pltpu.async_copy(src(1), ib1, si1)
        nt = nch // 2

        def pair_body(t, carry):
            c0 = 2 * t

            @pl.when(t > 0)
            def _():
                pltpu.make_async_copy(ob0, dst(c0), so0).wait()
            pltpu.make_async_copy(src(c0), ib0, si0).wait()
            compute(ib0, ob0)
            pltpu.async_copy(ob0, dst(c0), so0)

            @pl.when(t + 1 < nt)
            def _():
                pltpu.async_copy(src(c0 + 2), ib0, si0)

            @pl.when(t > 0)
            def _():
                pltpu.make_async_copy(ob1, dst(c0 + 1), so1).wait()
            pltpu.make_async_copy(src(c0 + 1), ib1, si1).wait()
            compute(ib1, ob1)
            pltpu.async_copy(ob1, dst(c0 + 1), so1)

            @pl.when(t + 1 < nt)
            def _():
                pltpu.async_copy(src(c0 + 3), ib1, si1)

            return carry

        lax.fori_loop(0, nt, pair_body, 0)
        pltpu.make_async_copy(ob0, dst(nch - 2), so0).wait()
        pltpu.make_async_copy(ob1, dst(nch - 1), so1).wait()

    return sc_group_sort


def kernel(x):
    b, f = x.shape
    return _build(b, f)(x)


# speedup vs baseline: 1.0249x; 1.0249x over previous
import functools

import jax
import jax.numpy as jnp
from jax import lax
from jax.experimental import pallas as pl
from jax.experimental.pallas import tpu as pltpu
from jax.experimental.pallas import tpu_sc as plsc

_GS = 128
_LANES = 16
_VPG = _GS // _LANES
_KG = 16
_U = 4


def _sort16(v):
    return lax.sort(v, dimension=0)


def _rev(v):
    return lax.rev(v, (0,))


def _merge(a, b):
    m = len(a)
    c = a + [_rev(b[m - 1 - i]) for i in range(m)]
    stride = m
    while stride >= 1:
        nxt = list(c)
        for base in range(0, 2 * m, 2 * stride):
            for i in range(stride):
                lo, hi = c[base + i], c[base + stride + i]
                nxt[base + i] = jnp.minimum(lo, hi)
                nxt[base + stride + i] = jnp.maximum(lo, hi)
        c = nxt
        stride //= 2
    return [_sort16(v) for v in c]


def _sort_group(vs):
    runs = [[_sort16(v)] for v in vs]
    while len(runs) > 1:
        runs = [_merge(runs[2 * i], runs[2 * i + 1])
                for i in range(len(runs) // 2)]
    return runs[0]


@functools.lru_cache(maxsize=None)
def _build(nbatch, nfeat):
    ngroups = nfeat // _GS
    info = plsc.get_sparse_core_info()
    nc = info.num_cores
    gpw = ngroups // 2
    nch = gpw // _KG
    mesh = plsc.VectorSubcoreMesh(core_axis_name="c", subcore_axis_name="s")

    w = _GS * _KG

    @functools.partial(
        pl.kernel,
        mesh=mesh,
        out_type=jax.ShapeDtypeStruct((nbatch, nfeat), jnp.float32),
        scratch_types=[
            pltpu.VMEM((8, w), jnp.float32),
            pltpu.VMEM((8, w), jnp.float32),
            pltpu.VMEM((8, w), jnp.float32),
            pltpu.VMEM((8, w), jnp.float32),
            pltpu.SemaphoreType.DMA,
            pltpu.SemaphoreType.DMA,
            pltpu.SemaphoreType.DMA,
            pltpu.SemaphoreType.DMA,
        ],
        compiler_params=pltpu.CompilerParams(needs_layout_passes=False),
    )
    def sc_group_sort(x_hbm, out_hbm, ib0, ib1, ob0, ob1, si0, si1, so0, so1):
        wid = lax.axis_index("s") * nc + lax.axis_index("c")
        trow = wid // 2
        ghalf = wid % 2
        r0 = trow * 8
        g0 = ghalf * gpw

        def src(c):
            return x_hbm.at[pl.ds(r0, 8), pl.ds((g0 + c * _KG) * _GS, w)]

        def dst(c):
            return out_hbm.at[pl.ds(r0, 8), pl.ds((g0 + c * _KG) * _GS, w)]

        def compute(ib, ob):
            @plsc.parallel_loop(0, 8 * _KG, step=1, unroll=_U)
            def body(idx):
                br = lax.rem(idx, 8)
                col = (idx // 8) * _GS
                vs = [ib[br, pl.ds(col + j * _LANES, _LANES)]
                      for j in range(_VPG)]
                sv = _sort_group(vs)
                for j in range(_VPG):
                    ob[br, pl.ds(col + j * _LANES, _LANES)] = sv[j]

        pltpu.async_copy(src(0), ib0, si0)
        pltpu.async_copy(src(1), ib1, si1)
        nt = nch // 2

        def pair_body(t, carry):
            c0 = 2 * t

            @pl.when(t > 0)
            def _():
                pltpu.make_async_copy(ob0, dst(c0), so0).wait()
            pltpu.make_async_copy(src(c0), ib0, si0).wait()
            compute(ib0, ob0)
            pltpu.async_copy(ob0, dst(c0), so0)

            @pl.when(t + 1 < nt)
            def _():
                pltpu.async_copy(src(c0 + 2), ib0, si0)

            @pl.when(t > 0)
            def _():
                pltpu.make_async_copy(ob1, dst(c0 + 1), so1).wait()
            pltpu.make_async_copy(src(c0 + 1), ib1, si1).wait()
            compute(ib1, ob1)
            pltpu.async_copy(ob1, dst(c0 + 1), so1)

            @pl.when(t + 1 < nt)
            def _():
                pltpu.async_copy(src(c0 + 3), ib1, si1)

            return carry

        lax.fori_loop(0, nt, pair_body, 0)
        pltpu.make_async_copy(ob0, dst(nch - 2), so0).wait()
        pltpu.make_async_copy(ob1, dst(nch - 1), so1).wait()

    return sc_group_sort


def kernel(x):
    b, f = x.shape
    return _build(b, f)(x)
